# baseline (speedup 1.0000x reference)
import jax
import jax.numpy as jnp
from jax.experimental import pallas as pl

B, N, IN_C, OUT_C, NB, KS = 4, 4096, 64, 64, 20, 8


def _bn_kernel(x_ref, g_ref, b_ref, o_ref):
    x = x_ref[...]
    mean = jnp.mean(x, axis=0, keepdims=True)
    var = jnp.mean((x - mean) ** 2, axis=0, keepdims=True)
    o_ref[...] = (x - mean) / jnp.sqrt(var + 1e-5) * g_ref[...] + b_ref[...]


def kernel(x, feature, mlp_w, mlp_b, conv_w, conv_b, kernals, one_padding, bn_gamma, bn_beta):
    bsize, _, num_pts = x.shape
    feats = feature.shape[1]
    inner = -2.0 * jnp.einsum('bcn,bcm->bnm', x, x)
    xx = jnp.sum(x ** 2, axis=1, keepdims=True)
    pairwise = -xx - inner - jnp.transpose(xx, (0, 2, 1))
    idx = jax.lax.top_k(pairwise, NB)[1]
    x_flat = jnp.transpose(x, (0, 2, 1)).reshape(bsize * num_pts, 3)
    idx_base = (jnp.arange(bsize) * num_pts)[:, None, None]
    flat_idx = (idx + idx_base).reshape(-1)
    x_spirals = jnp.take(x_flat, flat_idx, axis=0).reshape(bsize * num_pts, NB, 3)
    x_rel = x_spirals - x_spirals[:, 0:1, :]
    x_dis = jnp.sqrt(jnp.sum(x_rel ** 2, axis=-1, keepdims=True) + 1e-12)
    x_feats = jnp.concatenate([jnp.broadcast_to(x_spirals[:, 0:1, :], x_spirals.shape), x_rel, x_dis], axis=-1)
    x_feats = jnp.einsum('nkc,ic->nik', x_feats, mlp_w) + mlp_b[None, :, None]
    f_flat = jnp.transpose(feature, (0, 2, 1)).reshape(bsize * num_pts, feats)
    spirals = jnp.transpose(jnp.take(f_flat, flat_idx, axis=0).reshape(bsize * num_pts, NB, feats), (0, 2, 1))
    spirals = jnp.concatenate([spirals, x_feats], axis=1)
    adjw = jnp.matmul(x_rel, kernals) + one_padding
    adjw = jnp.where(adjw > 0, adjw, 0.0)
    adjw = adjw / (jnp.sum(adjw, axis=1, keepdims=True) + 1e-6)
    adjw = adjw * adjw
    adjw = adjw / (jnp.sum(adjw, axis=1, keepdims=True) + 1e-6)
    adjw = jnp.where(adjw > 0.1, adjw, 0.0)
    spirals = jnp.matmul(spirals, adjw).reshape(bsize * num_pts, feats * 2 * KS)
    out = spirals @ conv_w.T + conv_b

    out = pl.pallas_call(
        _bn_kernel,
        out_shape=jax.ShapeDtypeStruct((bsize * num_pts, OUT_C), jnp.float32),
    )(out, bn_gamma[None, :], bn_beta[None, :])

    out = jnp.transpose(out.reshape(bsize, num_pts, OUT_C), (0, 2, 1))
    return out


# speedup vs baseline: 3.9742x; 3.9742x over previous
import jax
import jax.numpy as jnp
from jax.experimental import pallas as pl

B, N, IN_C, OUT_C, NB, KS = 4, 4096, 64, 64, 20, 8
Q = 256


def _knn_body(q_ref, c_ref, o_ref):
    b = pl.program_id(0)
    q = q_ref[0]
    c = c_ref[0]
    dn = (((0,), (0,)), ((), ()))
    qc = jax.lax.dot_general(q, c, dn, preferred_element_type=jnp.float32)
    qq = jax.lax.dot_general(q * q, jnp.ones((3, 1), jnp.float32), dn,
                             preferred_element_type=jnp.float32)
    cc = jnp.sum(c * c, axis=0, keepdims=True)
    vals = 2.0 * qc - qq - cc
    iota = jax.lax.broadcasted_iota(jnp.int32, (Q, N), 1)
    cols = []
    for _ in range(NB):
        m = jnp.max(vals, axis=1, keepdims=True)
        mi = jnp.where(vals == m, iota, N)
        am = jnp.min(mi, axis=1, keepdims=True)
        cols.append(am)
        vals = jnp.where(mi == am, jnp.float32(-jnp.inf), vals)
    o_ref[0] = jnp.concatenate(cols, axis=1) + b * N


def _knn(x):
    return pl.pallas_call(
        _knn_body,
        grid=(B, N // Q),
        in_specs=[
            pl.BlockSpec((1, 3, Q), lambda b, i: (b, 0, i)),
            pl.BlockSpec((1, 3, N), lambda b, i: (b, 0, 0)),
        ],
        out_specs=pl.BlockSpec((1, Q, NB), lambda b, i: (b, i, 0)),
        out_shape=jax.ShapeDtypeStruct((B, N, NB), jnp.int32),
    )(x, x)


def _bn_kernel(x_ref, g_ref, b_ref, o_ref):
    x = x_ref[...]
    mean = jnp.mean(x, axis=0, keepdims=True)
    var = jnp.mean((x - mean) ** 2, axis=0, keepdims=True)
    o_ref[...] = (x - mean) / jnp.sqrt(var + 1e-5) * g_ref[...] + b_ref[...]


def kernel(x, feature, mlp_w, mlp_b, conv_w, conv_b, kernals, one_padding, bn_gamma, bn_beta):
    bsize, _, num_pts = x.shape
    feats = feature.shape[1]
    flat_idx = _knn(x).reshape(-1)
    x_flat = jnp.transpose(x, (0, 2, 1)).reshape(bsize * num_pts, 3)
    x_spirals = jnp.take(x_flat, flat_idx, axis=0).reshape(bsize * num_pts, NB, 3)
    x_rel = x_spirals - x_spirals[:, 0:1, :]
    x_dis = jnp.sqrt(jnp.sum(x_rel ** 2, axis=-1, keepdims=True) + 1e-12)
    x_feats = jnp.concatenate([jnp.broadcast_to(x_spirals[:, 0:1, :], x_spirals.shape), x_rel, x_dis], axis=-1)
    x_feats = jnp.einsum('nkc,ic->nik', x_feats, mlp_w) + mlp_b[None, :, None]
    f_flat = jnp.transpose(feature, (0, 2, 1)).reshape(bsize * num_pts, feats)
    spirals = jnp.transpose(jnp.take(f_flat, flat_idx, axis=0).reshape(bsize * num_pts, NB, feats), (0, 2, 1))
    spirals = jnp.concatenate([spirals, x_feats], axis=1)
    adjw = jnp.matmul(x_rel, kernals) + one_padding
    adjw = jnp.where(adjw > 0, adjw, 0.0)
    adjw = adjw / (jnp.sum(adjw, axis=1, keepdims=True) + 1e-6)
    adjw = adjw * adjw
    adjw = adjw / (jnp.sum(adjw, axis=1, keepdims=True) + 1e-6)
    adjw = jnp.where(adjw > 0.1, adjw, 0.0)
    spirals = jnp.matmul(spirals, adjw).reshape(bsize * num_pts, feats * 2 * KS)
    out = spirals @ conv_w.T + conv_b

    out = pl.pallas_call(
        _bn_kernel,
        out_shape=jax.ShapeDtypeStruct((bsize * num_pts, OUT_C), jnp.float32),
    )(out, bn_gamma[None, :], bn_beta[None, :])

    out = jnp.transpose(out.reshape(bsize, num_pts, OUT_C), (0, 2, 1))
    return out


# speedup vs baseline: 6.4405x; 1.6206x over previous
import functools

import jax
import jax.numpy as jnp
from jax import lax
from jax.experimental import pallas as pl
from jax.experimental.pallas import tpu as pltpu
from jax.experimental.pallas import tpu_sc as plsc

B, N, IN_C, OUT_C, NB, KS = 4, 4096, 64, 64, 20, 8
BN = B * N
Q = 256
P_AGG = 256
NW = 32
CH = 128


def _knn_body(q_ref, c_ref, o_ref):
    b = pl.program_id(0)
    q = q_ref[0]
    c = c_ref[0]
    dn = (((0,), (0,)), ((), ()))
    qc = lax.dot_general(q, c, dn, preferred_element_type=jnp.float32)
    qq = lax.dot_general(q * q, jnp.ones((3, 1), jnp.float32), dn,
                         preferred_element_type=jnp.float32)
    cc = jnp.sum(c * c, axis=0, keepdims=True)
    vals = 2.0 * qc - qq - cc
    iota = lax.broadcasted_iota(jnp.int32, (Q, N), 1)
    cols = []
    for _ in range(NB):
        m = jnp.max(vals, axis=1, keepdims=True)
        mi = jnp.where(vals == m, iota, N)
        am = jnp.min(mi, axis=1, keepdims=True)
        cols.append(am)
        vals = jnp.where(mi == am, jnp.float32(-jnp.inf), vals)
    o_ref[0] = jnp.concatenate(cols, axis=1) + b * N


def _knn(x):
    return pl.pallas_call(
        _knn_body,
        grid=(B, N // Q),
        in_specs=[
            pl.BlockSpec((1, 3, Q), lambda b, i: (b, 0, i)),
            pl.BlockSpec((1, 3, N), lambda b, i: (b, 0, 0)),
        ],
        out_specs=pl.BlockSpec((1, Q, NB), lambda b, i: (b, i, 0)),
        out_shape=jax.ShapeDtypeStruct((B, N, NB), jnp.int32),
    )(x, x)


def _sc_gather(tab, fidx):
    per_w = (BN * NB) // NW
    steps = per_w // CH
    mesh = plsc.VectorSubcoreMesh(core_axis_name="c", subcore_axis_name="s")

    @functools.partial(
        pl.kernel, mesh=mesh,
        out_type=jax.ShapeDtypeStruct((BN * NB, 128), jnp.float32),
        scratch_types=[
            pltpu.VMEM((CH,), jnp.int32),
            pltpu.VMEM((CH, 128), jnp.float32),
            pltpu.SemaphoreType.DMA,
        ],
    )
    def gather_k(tab_h, idx_h, g_h, idx_v, fbuf, s1):
        wid = lax.axis_index("s") * 2 + lax.axis_index("c")

        def body(t, carry):
            base = wid * per_w + t * CH
            pltpu.sync_copy(idx_h.at[pl.ds(base, CH)], idx_v)
            pltpu.async_copy(tab_h.at[idx_v], fbuf, s1).wait()
            pltpu.sync_copy(fbuf, g_h.at[pl.ds(base, CH)])
            return carry

        lax.fori_loop(0, steps, body, 0)

    return gather_k(tab, fidx)


def _agg_body(g_ref, kern_ref, op_ref, w1_ref, w2_ref, cb_ref, o_ref):
    kern = kern_ref[...]
    opt = op_ref[...]
    rel, cent = [], []
    for d in range(3):
        cols = [g_ref[:, j * 128 + IN_C + d:j * 128 + IN_C + d + 1] for j in range(NB)]
        row = jnp.concatenate(cols, axis=1)
        c0 = row[:, 0:1]
        cent.append(c0)
        rel.append(row - c0)
    dis = jnp.sqrt(rel[0] * rel[0] + rel[1] * rel[1] + rel[2] * rel[2] + 1e-12)
    ones = jnp.ones_like(dis)
    shape = dis.shape
    xf = [jnp.broadcast_to(cent[0], shape), jnp.broadcast_to(cent[1], shape),
          jnp.broadcast_to(cent[2], shape), rel[0], rel[1], rel[2], dis, ones]
    xf_cat = jnp.concatenate(xf, axis=1)

    relb = [r.astype(jnp.bfloat16).astype(jnp.float32) for r in rel]
    kernb = kern.astype(jnp.bfloat16).astype(jnp.float32)
    a_ks = []
    for k in range(KS):
        a = (relb[0] * kernb[0:1, k:k + 1] + relb[1] * kernb[1:2, k:k + 1]
             + relb[2] * kernb[2:3, k:k + 1] + opt[k:k + 1, :])
        a = jnp.maximum(a, 0.0)
        a = a / (jnp.sum(a, axis=1, keepdims=True) + 1e-6)
        a = a * a
        a = a / (jnp.sum(a, axis=1, keepdims=True) + 1e-6)
        a = jnp.where(a > 0.1, a, 0.0)
        a_ks.append(a)

    t_cat = jnp.concatenate(
        [xf_cat * jnp.concatenate([a_ks[k]] * 8, axis=1) for k in range(KS)],
        axis=1)

    accs = [None] * KS
    for j in range(NB):
        gj = g_ref[:, j * 128:j * 128 + IN_C]
        for k in range(KS):
            term = gj * a_ks[k][:, j:j + 1]
            accs[k] = term if accs[k] is None else accs[k] + term
    m_cat = jnp.concatenate(accs, axis=1)

    dn = (((1,), (0,)), ((), ()))
    out = lax.dot_general(m_cat, w1_ref[...], dn, preferred_element_type=jnp.float32)
    out = out + lax.dot_general(t_cat, w2_ref[...], dn, preferred_element_type=jnp.float32)
    o_ref[...] = out + cb_ref[...]


def _agg(gm, kernals, op_t, w1, w2e, cb):
    return pl.pallas_call(
        _agg_body,
        grid=(BN // P_AGG,),
        in_specs=[
            pl.BlockSpec((P_AGG, NB * 128), lambda i: (i, 0)),
            pl.BlockSpec((3, KS), lambda i: (0, 0)),
            pl.BlockSpec((KS, NB), lambda i: (0, 0)),
            pl.BlockSpec((KS * IN_C, OUT_C), lambda i: (0, 0)),
            pl.BlockSpec((KS * 8 * NB, OUT_C), lambda i: (0, 0)),
            pl.BlockSpec((1, OUT_C), lambda i: (0, 0)),
        ],
        out_specs=pl.BlockSpec((P_AGG, OUT_C), lambda i: (i, 0)),
        out_shape=jax.ShapeDtypeStruct((BN, OUT_C), jnp.float32),
    )(gm, kernals, op_t, w1, w2e, cb)


def _bn_kernel(x_ref, g_ref, b_ref, o_ref):
    x = x_ref[...]
    mean = jnp.mean(x, axis=0, keepdims=True)
    var = jnp.mean((x - mean) ** 2, axis=0, keepdims=True)
    o_ref[...] = (x - mean) / jnp.sqrt(var + 1e-5) * g_ref[...] + b_ref[...]


def kernel(x, feature, mlp_w, mlp_b, conv_w, conv_b, kernals, one_padding, bn_gamma, bn_beta):
    fidx = _knn(x).reshape(-1)

    f_flat = jnp.transpose(feature, (0, 2, 1)).reshape(BN, IN_C)
    x_flat = jnp.transpose(x, (0, 2, 1)).reshape(BN, 3)
    tab = jnp.concatenate([f_flat, x_flat, jnp.zeros((BN, 128 - IN_C - 3), jnp.float32)], axis=1)

    g2 = _sc_gather(tab, fidx)
    gm = g2.reshape(BN, NB * 128)

    cw = conv_w.reshape(OUT_C, 2 * IN_C, KS)
    w1 = cw[:, :IN_C, :].transpose(2, 1, 0).reshape(KS * IN_C, OUT_C)
    mlp_aug = jnp.concatenate([mlp_w, mlp_b[:, None]], axis=1)
    w2 = jnp.einsum('oik,iq->kqo', cw[:, IN_C:, :], mlp_aug)
    w2e = jnp.broadcast_to(w2[:, :, None, :], (KS, 8, NB, OUT_C)).reshape(KS * 8 * NB, OUT_C)
    op_t = one_padding.T

    raw = _agg(gm, kernals, op_t, w1, w2e, conv_b[None, :])

    out = pl.pallas_call(
        _bn_kernel,
        out_shape=jax.ShapeDtypeStruct((BN, OUT_C), jnp.float32),
    )(raw, bn_gamma[None, :], bn_beta[None, :])

    return jnp.transpose(out.reshape(B, N, OUT_C), (0, 2, 1))
